# Initial kernel scaffold; baseline (speedup 1.0000x reference)
import jax
import jax.numpy as jnp
from jax.experimental import pallas as pl


def kernel(x, edge_index, W1, b1, W2, b2, W3, b3, W4, b4):
    raise NotImplementedError("write your pallas kernel here")


# speedup vs baseline: 18.7777x; 18.7777x over previous
import functools

import jax
import jax.numpy as jnp
from jax import lax
from jax.experimental import pallas as pl
from jax.experimental.pallas import tpu as pltpu
from jax.experimental.pallas import tpu_sc as plsc

N = 10000
N_PAD = 10240
E = 160000
NC, NS = 2, 16
NW = NC * NS
EPT = E // NW
CHUNK = 125
NCHUNK = EPT // CHUNK
STRIPE = N_PAD // NS
BLK = 1024


def _sc_scatter_body(g_hbm, srci_hbm, dsti_hbm, zrows_hbm, out_hbm,
                     srcv, dstv, rows, acc, sem, *, feat):
    c = lax.axis_index("c")
    s = lax.axis_index("s")
    gwid = c * NS + s

    pltpu.sync_copy(zrows_hbm, acc.at[pl.ds(s * STRIPE, STRIPE)])
    pltpu.sync_copy(srci_hbm.at[gwid], srcv)
    pltpu.sync_copy(dsti_hbm.at[gwid], dstv)
    plsc.subcore_barrier()

    def body(j, carry):
        pltpu.async_copy(g_hbm.at[srcv.at[j]], rows, sem).wait()
        pltpu.sync_copy(rows, acc.at[dstv.at[j]], add=True)
        return carry

    lax.fori_loop(0, NCHUNK, body, 0, unroll=False)
    plsc.subcore_barrier()
    pltpu.sync_copy(acc.at[pl.ds(s * STRIPE, STRIPE)],
                    out_hbm.at[c].at[pl.ds(s * STRIPE, STRIPE)])


def _make_sc_scatter(feat):
    mesh = plsc.VectorSubcoreMesh(core_axis_name="c", subcore_axis_name="s")
    return functools.partial(
        pl.kernel,
        out_type=jax.ShapeDtypeStruct((NC, N_PAD, feat), jnp.float32),
        mesh=mesh,
        scratch_types=[
            pltpu.VMEM((NCHUNK, CHUNK), jnp.int32),
            pltpu.VMEM((NCHUNK, CHUNK), jnp.int32),
            pltpu.VMEM((CHUNK, feat), jnp.float32),
            pltpu.VMEM_SHARED((N_PAD, feat), jnp.float32),
            pltpu.SemaphoreType.DMA,
        ],
        compiler_params=pltpu.CompilerParams(use_tc_tiling_on_sc=False),
        name=f"gcn_sc_scatter_{feat}",
    )(functools.partial(_sc_scatter_body, feat=feat))


def _sc_degree_body(ones_hbm, dsti_hbm, zrows_hbm, out_hbm,
                    dstv, rows, acc, sem):
    c = lax.axis_index("c")
    s = lax.axis_index("s")
    gwid = c * NS + s

    pltpu.sync_copy(zrows_hbm, acc.at[pl.ds(s * STRIPE, STRIPE)])
    pltpu.sync_copy(dsti_hbm.at[gwid], dstv)
    pltpu.sync_copy(ones_hbm, rows)
    plsc.subcore_barrier()

    def body(j, carry):
        pltpu.sync_copy(rows, acc.at[dstv.at[j]], add=True)
        return carry

    lax.fori_loop(0, NCHUNK, body, 0, unroll=False)
    plsc.subcore_barrier()
    pltpu.sync_copy(acc.at[pl.ds(s * STRIPE, STRIPE)],
                    out_hbm.at[c].at[pl.ds(s * STRIPE, STRIPE)])


def _make_sc_degree():
    mesh = plsc.VectorSubcoreMesh(core_axis_name="c", subcore_axis_name="s")
    return functools.partial(
        pl.kernel,
        out_type=jax.ShapeDtypeStruct((NC, N_PAD, 16), jnp.float32),
        mesh=mesh,
        scratch_types=[
            pltpu.VMEM((NCHUNK, CHUNK), jnp.int32),
            pltpu.VMEM((CHUNK, 16), jnp.float32),
            pltpu.VMEM_SHARED((N_PAD, 16), jnp.float32),
            pltpu.SemaphoreType.DMA,
        ],
        compiler_params=pltpu.CompilerParams(use_tc_tiling_on_sc=False),
        name="gcn_sc_degree",
    )(_sc_degree_body)


def _dinv_of(deg_ref):
    dg = deg_ref[0, :, 0] + deg_ref[1, :, 0] + 1.0
    return lax.rsqrt(dg)[:, None]


def _tc_first_body(x_ref, w_ref, deg_ref, g_ref):
    h = jnp.dot(x_ref[...], w_ref[...], preferred_element_type=jnp.float32)
    g_ref[...] = h * _dinv_of(deg_ref)


def _tc_first(x_pad, W1, degacc):
    return pl.pallas_call(
        _tc_first_body,
        grid=(N_PAD // BLK,),
        in_specs=[
            pl.BlockSpec((BLK, 256), lambda i: (i, 0)),
            pl.BlockSpec((256, 64), lambda i: (0, 0)),
            pl.BlockSpec((2, BLK, 16), lambda i: (0, i, 0)),
        ],
        out_specs=pl.BlockSpec((BLK, 64), lambda i: (i, 0)),
        out_shape=jax.ShapeDtypeStruct((N_PAD, 64), jnp.float32),
    )(x_pad, W1, degacc)


def _tc_mid_body(acc_ref, g_ref, deg_ref, b_ref, w_ref, out_ref):
    dinv = _dinv_of(deg_ref)
    z = (acc_ref[0] + acc_ref[1] + g_ref[...]) * dinv + b_ref[...]
    z = jnp.maximum(z, 0.0)
    out_ref[...] = jnp.dot(z, w_ref[...],
                           preferred_element_type=jnp.float32) * dinv


def _tc_mid(accparts, g_prev, degacc, b, W, h_in, h_out):
    return pl.pallas_call(
        _tc_mid_body,
        grid=(N_PAD // BLK,),
        in_specs=[
            pl.BlockSpec((2, BLK, h_in), lambda i: (0, i, 0)),
            pl.BlockSpec((BLK, h_in), lambda i: (i, 0)),
            pl.BlockSpec((2, BLK, 16), lambda i: (0, i, 0)),
            pl.BlockSpec((1, h_in), lambda i: (0, 0)),
            pl.BlockSpec((h_in, h_out), lambda i: (0, 0)),
        ],
        out_specs=pl.BlockSpec((BLK, h_out), lambda i: (i, 0)),
        out_shape=jax.ShapeDtypeStruct((N_PAD, h_out), jnp.float32),
    )(accparts, g_prev, degacc, b.reshape(1, h_in), W)


def _tc_final_body(acc_ref, g_ref, deg_ref, b3_ref, w4_ref, b4_ref, out_ref):
    dinv = _dinv_of(deg_ref)
    z = (acc_ref[0] + acc_ref[1] + g_ref[...]) * dinv + b3_ref[...]
    z = jnp.maximum(z, 0.0)
    out_ref[...] = jnp.dot(z, w4_ref[...],
                           preferred_element_type=jnp.float32) + b4_ref[...]


def _tc_final(accparts, g3, degacc, b3, W4, b4):
    return pl.pallas_call(
        _tc_final_body,
        grid=(N_PAD // BLK,),
        in_specs=[
            pl.BlockSpec((2, BLK, 16), lambda i: (0, i, 0)),
            pl.BlockSpec((BLK, 16), lambda i: (i, 0)),
            pl.BlockSpec((2, BLK, 16), lambda i: (0, i, 0)),
            pl.BlockSpec((1, 16), lambda i: (0, 0)),
            pl.BlockSpec((16, 2), lambda i: (0, 0)),
            pl.BlockSpec((1, 2), lambda i: (0, 0)),
        ],
        out_specs=pl.BlockSpec((BLK, 2), lambda i: (i, 0)),
        out_shape=jax.ShapeDtypeStruct((N_PAD, 2), jnp.float32),
    )(accparts, g3, degacc, b3.reshape(1, 16), W4, b4.reshape(1, 2))


@jax.jit
def kernel(x, edge_index, W1, b1, W2, b2, W3, b3, W4, b4):
    src = edge_index[0].astype(jnp.int32).reshape(NW, NCHUNK, CHUNK)
    dst = edge_index[1].astype(jnp.int32).reshape(NW, NCHUNK, CHUNK)
    x_pad = jnp.zeros((N_PAD, 256), jnp.float32).at[:N].set(x)

    ones_rows = jnp.ones((CHUNK, 16), jnp.float32)
    z16 = jnp.zeros((STRIPE, 16), jnp.float32)
    degacc = _make_sc_degree()(ones_rows, dst, z16)

    g1 = _tc_first(x_pad, W1, degacc)
    acc1 = _make_sc_scatter(64)(g1, src, dst, jnp.zeros((STRIPE, 64), jnp.float32))
    g2 = _tc_mid(acc1, g1, degacc, b1, W2, 64, 32)
    acc2 = _make_sc_scatter(32)(g2, src, dst, jnp.zeros((STRIPE, 32), jnp.float32))
    g3 = _tc_mid(acc2, g2, degacc, b2, W3, 32, 16)
    acc3 = _make_sc_scatter(16)(g3, src, dst, z16)
    logits = _tc_final(acc3, g3, degacc, b3, W4, b4)
    return logits[:N]


# speedup vs baseline: 25.3234x; 1.3486x over previous
import functools

import jax
import jax.numpy as jnp
from jax import lax
from jax.experimental import pallas as pl
from jax.experimental.pallas import tpu as pltpu
from jax.experimental.pallas import tpu_sc as plsc

N = 10000
N_PAD = 10240
E = 160000
NC, NS = 2, 16
NW = NC * NS
EPT = E // NW
CHUNK = 125
NCHUNK = EPT // CHUNK
STRIPE = N_PAD // NS
BLK = 1024


RING = 4
LOOK = 2


def _sc_scatter_body(g_hbm, srci_hbm, dsti_hbm, zrows_hbm, out_hbm,
                     srcv, dstv, rows, acc, gsem, ssem, *, feat):
    c = lax.axis_index("c")
    s = lax.axis_index("s")
    gwid = c * NS + s

    pltpu.sync_copy(zrows_hbm, acc.at[pl.ds(s * STRIPE, STRIPE)])
    pltpu.sync_copy(srci_hbm.at[gwid], srcv)
    pltpu.sync_copy(dsti_hbm.at[gwid], dstv)
    plsc.subcore_barrier()

    def rowslot(j):
        r = lax.rem(j, RING)
        return rows.at[pl.ds(r * CHUNK, CHUNK)]

    pltpu.async_copy(g_hbm.at[srcv.at[0]], rowslot(0), gsem)
    pltpu.async_copy(g_hbm.at[srcv.at[1]], rowslot(1), gsem)

    def body(j, carry):
        q = j + LOOK

        @pl.when(q < NCHUNK)
        def _prefetch():
            @pl.when(j >= LOOK)
            def _():
                pltpu.make_async_copy(rowslot(j), acc.at[dstv.at[j]],
                                      ssem).wait()
            pltpu.async_copy(g_hbm.at[srcv.at[q]], rowslot(q), gsem)

        pltpu.make_async_copy(g_hbm.at[srcv.at[j]], rowslot(j), gsem).wait()
        pltpu.async_copy(rowslot(j), acc.at[dstv.at[j]], ssem, add=True)
        return carry

    lax.fori_loop(0, NCHUNK, body, 0, unroll=False)

    def drain(j, carry):
        pltpu.make_async_copy(rowslot(0), acc.at[dstv.at[0]], ssem).wait()
        return carry

    lax.fori_loop(0, RING, drain, 0, unroll=False)
    plsc.subcore_barrier()
    pltpu.sync_copy(acc.at[pl.ds(s * STRIPE, STRIPE)],
                    out_hbm.at[c].at[pl.ds(s * STRIPE, STRIPE)])


def _make_sc_scatter(feat):
    mesh = plsc.VectorSubcoreMesh(core_axis_name="c", subcore_axis_name="s")
    return functools.partial(
        pl.kernel,
        out_type=jax.ShapeDtypeStruct((NC, N_PAD, feat), jnp.float32),
        mesh=mesh,
        scratch_types=[
            pltpu.VMEM((NCHUNK, CHUNK), jnp.int32),
            pltpu.VMEM((NCHUNK, CHUNK), jnp.int32),
            pltpu.VMEM((RING * CHUNK, feat), jnp.float32),
            pltpu.VMEM_SHARED((N_PAD, feat), jnp.float32),
            pltpu.SemaphoreType.DMA,
            pltpu.SemaphoreType.DMA,
        ],
        compiler_params=pltpu.CompilerParams(use_tc_tiling_on_sc=False),
        name=f"gcn_sc_scatter_{feat}",
    )(functools.partial(_sc_scatter_body, feat=feat))


def _sc_degree_body(ones_hbm, dsti_hbm, zrows_hbm, out_hbm,
                    dstv, rows, acc, sem):
    c = lax.axis_index("c")
    s = lax.axis_index("s")
    gwid = c * NS + s

    pltpu.sync_copy(zrows_hbm, acc.at[pl.ds(s * STRIPE, STRIPE)])
    pltpu.sync_copy(dsti_hbm.at[gwid], dstv)
    pltpu.sync_copy(ones_hbm, rows)
    plsc.subcore_barrier()

    def body(j, carry):
        pltpu.async_copy(rows, acc.at[dstv.at[j]], sem, add=True)
        return carry

    lax.fori_loop(0, NCHUNK, body, 0, unroll=False)

    def drain(j, carry):
        pltpu.make_async_copy(rows, acc.at[dstv.at[0]], sem).wait()
        return carry

    lax.fori_loop(0, NCHUNK, drain, 0, unroll=False)
    plsc.subcore_barrier()
    pltpu.sync_copy(acc.at[pl.ds(s * STRIPE, STRIPE)],
                    out_hbm.at[c].at[pl.ds(s * STRIPE, STRIPE)])


def _make_sc_degree():
    mesh = plsc.VectorSubcoreMesh(core_axis_name="c", subcore_axis_name="s")
    return functools.partial(
        pl.kernel,
        out_type=jax.ShapeDtypeStruct((NC, N_PAD, 16), jnp.float32),
        mesh=mesh,
        scratch_types=[
            pltpu.VMEM((NCHUNK, CHUNK), jnp.int32),
            pltpu.VMEM((CHUNK, 16), jnp.float32),
            pltpu.VMEM_SHARED((N_PAD, 16), jnp.float32),
            pltpu.SemaphoreType.DMA,
        ],
        compiler_params=pltpu.CompilerParams(use_tc_tiling_on_sc=False),
        name="gcn_sc_degree",
    )(_sc_degree_body)


def _dinv_of(deg_ref):
    dg = deg_ref[0, :, 0] + deg_ref[1, :, 0] + 1.0
    return lax.rsqrt(dg)[:, None]


def _tc_first_body(x_ref, w_ref, deg_ref, g_ref):
    h = jnp.dot(x_ref[...], w_ref[...], preferred_element_type=jnp.float32)
    g_ref[...] = h * _dinv_of(deg_ref)


def _tc_first(x_pad, W1, degacc):
    return pl.pallas_call(
        _tc_first_body,
        grid=(N_PAD // BLK,),
        in_specs=[
            pl.BlockSpec((BLK, 256), lambda i: (i, 0)),
            pl.BlockSpec((256, 64), lambda i: (0, 0)),
            pl.BlockSpec((2, BLK, 16), lambda i: (0, i, 0)),
        ],
        out_specs=pl.BlockSpec((BLK, 64), lambda i: (i, 0)),
        out_shape=jax.ShapeDtypeStruct((N_PAD, 64), jnp.float32),
    )(x_pad, W1, degacc)


def _tc_mid_body(acc_ref, g_ref, deg_ref, b_ref, w_ref, out_ref):
    dinv = _dinv_of(deg_ref)
    z = (acc_ref[0] + acc_ref[1] + g_ref[...]) * dinv + b_ref[...]
    z = jnp.maximum(z, 0.0)
    out_ref[...] = jnp.dot(z, w_ref[...],
                           preferred_element_type=jnp.float32) * dinv


def _tc_mid(accparts, g_prev, degacc, b, W, h_in, h_out):
    return pl.pallas_call(
        _tc_mid_body,
        grid=(N_PAD // BLK,),
        in_specs=[
            pl.BlockSpec((2, BLK, h_in), lambda i: (0, i, 0)),
            pl.BlockSpec((BLK, h_in), lambda i: (i, 0)),
            pl.BlockSpec((2, BLK, 16), lambda i: (0, i, 0)),
            pl.BlockSpec((1, h_in), lambda i: (0, 0)),
            pl.BlockSpec((h_in, h_out), lambda i: (0, 0)),
        ],
        out_specs=pl.BlockSpec((BLK, h_out), lambda i: (i, 0)),
        out_shape=jax.ShapeDtypeStruct((N_PAD, h_out), jnp.float32),
    )(accparts, g_prev, degacc, b.reshape(1, h_in), W)


def _tc_final_body(acc_ref, g_ref, deg_ref, b3_ref, w4_ref, b4_ref, out_ref):
    dinv = _dinv_of(deg_ref)
    z = (acc_ref[0] + acc_ref[1] + g_ref[...]) * dinv + b3_ref[...]
    z = jnp.maximum(z, 0.0)
    out_ref[...] = jnp.dot(z, w4_ref[...],
                           preferred_element_type=jnp.float32) + b4_ref[...]


def _tc_final(accparts, g3, degacc, b3, W4, b4):
    return pl.pallas_call(
        _tc_final_body,
        grid=(N_PAD // BLK,),
        in_specs=[
            pl.BlockSpec((2, BLK, 16), lambda i: (0, i, 0)),
            pl.BlockSpec((BLK, 16), lambda i: (i, 0)),
            pl.BlockSpec((2, BLK, 16), lambda i: (0, i, 0)),
            pl.BlockSpec((1, 16), lambda i: (0, 0)),
            pl.BlockSpec((16, 2), lambda i: (0, 0)),
            pl.BlockSpec((1, 2), lambda i: (0, 0)),
        ],
        out_specs=pl.BlockSpec((BLK, 2), lambda i: (i, 0)),
        out_shape=jax.ShapeDtypeStruct((N_PAD, 2), jnp.float32),
    )(accparts, g3, degacc, b3.reshape(1, 16), W4, b4.reshape(1, 2))


@jax.jit
def kernel(x, edge_index, W1, b1, W2, b2, W3, b3, W4, b4):
    src = edge_index[0].astype(jnp.int32).reshape(NW, NCHUNK, CHUNK)
    dst = edge_index[1].astype(jnp.int32).reshape(NW, NCHUNK, CHUNK)
    x_pad = jnp.zeros((N_PAD, 256), jnp.float32).at[:N].set(x)

    ones_rows = jnp.ones((CHUNK, 16), jnp.float32)
    z16 = jnp.zeros((STRIPE, 16), jnp.float32)
    degacc = _make_sc_degree()(ones_rows, dst, z16)

    g1 = _tc_first(x_pad, W1, degacc)
    acc1 = _make_sc_scatter(64)(g1, src, dst, jnp.zeros((STRIPE, 64), jnp.float32))
    g2 = _tc_mid(acc1, g1, degacc, b1, W2, 64, 32)
    acc2 = _make_sc_scatter(32)(g2, src, dst, jnp.zeros((STRIPE, 32), jnp.float32))
    g3 = _tc_mid(acc2, g2, degacc, b2, W3, 32, 16)
    acc3 = _make_sc_scatter(16)(g3, src, dst, z16)
    logits = _tc_final(acc3, g3, degacc, b3, W4, b4)
    return logits[:N]


# speedup vs baseline: 27.2273x; 1.0752x over previous
import functools

import jax
import jax.numpy as jnp
from jax import lax
from jax.experimental import pallas as pl
from jax.experimental.pallas import tpu as pltpu
from jax.experimental.pallas import tpu_sc as plsc

N = 10000
N_PAD = 10240
E = 160000
NC, NS = 2, 16
NW = NC * NS
EPT = E // NW
CHUNK = 125
NCHUNK = EPT // CHUNK
STRIPE = N_PAD // NS
RING = 4
LOOK = 2

_SC_PARAMS = pltpu.CompilerParams(use_tc_tiling_on_sc=False)


def _sc_scatter_body(g_hbm, srci_hbm, dsti_hbm, zrows_hbm, out_hbm,
                     srcv, dstv, rows, acc, gsem, ssem, *, feat):
    c = lax.axis_index("c")
    s = lax.axis_index("s")
    gwid = c * NS + s

    pltpu.sync_copy(zrows_hbm, acc.at[pl.ds(s * STRIPE, STRIPE)])
    pltpu.sync_copy(srci_hbm.at[gwid], srcv)
    pltpu.sync_copy(dsti_hbm.at[gwid], dstv)
    plsc.subcore_barrier()

    def rowslot(j):
        r = lax.rem(j, RING)
        return rows.at[pl.ds(r * CHUNK, CHUNK)]

    pltpu.async_copy(g_hbm.at[srcv.at[0]], rowslot(0), gsem)
    pltpu.async_copy(g_hbm.at[srcv.at[1]], rowslot(1), gsem)

    def body(j, carry):
        q = j + LOOK

        @pl.when(q < NCHUNK)
        def _prefetch():
            @pl.when(j >= LOOK)
            def _():
                pltpu.make_async_copy(rowslot(j), acc.at[dstv.at[j]],
                                      ssem).wait()
            pltpu.async_copy(g_hbm.at[srcv.at[q]], rowslot(q), gsem)

        pltpu.make_async_copy(g_hbm.at[srcv.at[j]], rowslot(j), gsem).wait()
        pltpu.async_copy(rowslot(j), acc.at[dstv.at[j]], ssem, add=True)
        return carry

    lax.fori_loop(0, NCHUNK, body, 0, unroll=False)

    def drain(j, carry):
        pltpu.make_async_copy(rowslot(0), acc.at[dstv.at[0]], ssem).wait()
        return carry

    lax.fori_loop(0, RING, drain, 0, unroll=False)
    plsc.subcore_barrier()
    pltpu.sync_copy(acc.at[pl.ds(s * STRIPE, STRIPE)],
                    out_hbm.at[c].at[pl.ds(s * STRIPE, STRIPE)])


def _make_sc_scatter(feat):
    mesh = plsc.VectorSubcoreMesh(core_axis_name="c", subcore_axis_name="s")
    return functools.partial(
        pl.kernel,
        out_type=jax.ShapeDtypeStruct((NC, N_PAD, feat), jnp.float32),
        mesh=mesh,
        scratch_types=[
            pltpu.VMEM((NCHUNK, CHUNK), jnp.int32),
            pltpu.VMEM((NCHUNK, CHUNK), jnp.int32),
            pltpu.VMEM((RING * CHUNK, feat), jnp.float32),
            pltpu.VMEM_SHARED((N_PAD, feat), jnp.float32),
            pltpu.SemaphoreType.DMA,
            pltpu.SemaphoreType.DMA,
        ],
        compiler_params=_SC_PARAMS,
        name=f"gcn_sc_scatter_{feat}",
    )(functools.partial(_sc_scatter_body, feat=feat))


def _sc_degree_body(ones_hbm, dsti_hbm, zrows_hbm, out_hbm,
                    dstv, rows, acc, sem):
    c = lax.axis_index("c")
    s = lax.axis_index("s")
    gwid = c * NS + s

    pltpu.sync_copy(zrows_hbm, acc.at[pl.ds(s * STRIPE, STRIPE)])
    pltpu.sync_copy(dsti_hbm.at[gwid], dstv)
    pltpu.sync_copy(ones_hbm, rows)
    plsc.subcore_barrier()

    def body(j, carry):
        pltpu.async_copy(rows, acc.at[dstv.at[j]], sem, add=True)
        return carry

    lax.fori_loop(0, NCHUNK, body, 0, unroll=False)

    def drain(j, carry):
        pltpu.make_async_copy(rows, acc.at[dstv.at[0]], sem).wait()
        return carry

    lax.fori_loop(0, NCHUNK, drain, 0, unroll=False)
    plsc.subcore_barrier()
    pltpu.sync_copy(acc.at[pl.ds(s * STRIPE, STRIPE)],
                    out_hbm.at[c].at[pl.ds(s * STRIPE, STRIPE)])


def _make_sc_degree():
    mesh = plsc.VectorSubcoreMesh(core_axis_name="c", subcore_axis_name="s")
    return functools.partial(
        pl.kernel,
        out_type=jax.ShapeDtypeStruct((NC, N_PAD, 16), jnp.float32),
        mesh=mesh,
        scratch_types=[
            pltpu.VMEM((NCHUNK, CHUNK), jnp.int32),
            pltpu.VMEM((CHUNK, 16), jnp.float32),
            pltpu.VMEM_SHARED((N_PAD, 16), jnp.float32),
            pltpu.SemaphoreType.DMA,
        ],
        compiler_params=_SC_PARAMS,
        name="gcn_sc_degree",
    )(_sc_degree_body)


def _tc_first_body(x_ref, w_ref, deg_ref, g_ref, dinv_ref):
    deg = deg_ref[0, :, 0] + deg_ref[1, :, 0] + 1.0
    dinv = lax.rsqrt(deg)[:, None]
    dinv_ref[...] = dinv
    h = jnp.dot(x_ref[...], w_ref[...], preferred_element_type=jnp.float32)
    g_ref[...] = h * dinv[:N]


def _tc_first(x, W1, degacc):
    return pl.pallas_call(
        _tc_first_body,
        out_shape=(jax.ShapeDtypeStruct((N, 64), jnp.float32),
                   jax.ShapeDtypeStruct((N_PAD, 1), jnp.float32)),
    )(x, W1, degacc)


def _tc_mid_body(acc_ref, g_ref, dinv_ref, b_ref, w_ref, out_ref):
    dinv = dinv_ref[:N]
    z = (acc_ref[0, :N, :] + acc_ref[1, :N, :] + g_ref[...]) * dinv + b_ref[...]
    z = jnp.maximum(z, 0.0)
    out_ref[...] = jnp.dot(z, w_ref[...],
                           preferred_element_type=jnp.float32) * dinv


def _tc_mid(accparts, g_prev, dinv, b, W, h_in, h_out):
    return pl.pallas_call(
        _tc_mid_body,
        out_shape=jax.ShapeDtypeStruct((N, h_out), jnp.float32),
    )(accparts, g_prev, dinv, b.reshape(1, h_in), W)


def _tc_final_body(acc_ref, g_ref, dinv_ref, b3_ref, w4_ref, b4_ref, out_ref):
    dinv = dinv_ref[:N]
    z = (acc_ref[0, :N, :] + acc_ref[1, :N, :] + g_ref[...]) * dinv + b3_ref[...]
    z = jnp.maximum(z, 0.0)
    out_ref[...] = jnp.dot(z, w4_ref[...],
                           preferred_element_type=jnp.float32) + b4_ref[...]


def _tc_final(accparts, g3, dinv, b3, W4, b4):
    return pl.pallas_call(
        _tc_final_body,
        out_shape=jax.ShapeDtypeStruct((N, 2), jnp.float32),
    )(accparts, g3, dinv, b3.reshape(1, 16), W4, b4.reshape(1, 2))


@jax.jit
def kernel(x, edge_index, W1, b1, W2, b2, W3, b3, W4, b4):
    src = edge_index[0].astype(jnp.int32).reshape(NW, NCHUNK, CHUNK)
    dst = edge_index[1].astype(jnp.int32).reshape(NW, NCHUNK, CHUNK)

    ones_rows = jnp.ones((CHUNK, 16), jnp.float32)
    z16 = jnp.zeros((STRIPE, 16), jnp.float32)
    degacc = _make_sc_degree()(ones_rows, dst, z16)

    g1, dinv = _tc_first(x, W1, degacc)
    acc1 = _make_sc_scatter(64)(g1, src, dst, jnp.zeros((STRIPE, 64), jnp.float32))
    g2 = _tc_mid(acc1, g1, dinv, b1, W2, 64, 32)
    acc2 = _make_sc_scatter(32)(g2, src, dst, jnp.zeros((STRIPE, 32), jnp.float32))
    g3 = _tc_mid(acc2, g2, dinv, b2, W3, 32, 16)
    acc3 = _make_sc_scatter(16)(g3, src, dst, jnp.zeros((STRIPE, 16), jnp.float32))
    return _tc_final(acc3, g3, dinv, b3, W4, b4)


# speedup vs baseline: 27.8819x; 1.0240x over previous
import functools

import jax
import jax.numpy as jnp
from jax import lax
from jax.experimental import pallas as pl
from jax.experimental.pallas import tpu as pltpu
from jax.experimental.pallas import tpu_sc as plsc

N = 10000
N_PAD = 10240
E = 160000
NC, NS = 2, 16
NW = NC * NS
EPT = E // NW
CHUNK = 125
NCHUNK = EPT // CHUNK
STRIPE = N_PAD // NS
RING = 6
LOOK = 4

_SC_PARAMS = pltpu.CompilerParams(use_tc_tiling_on_sc=False)


def _sc_scatter_body(g_hbm, srci_hbm, dsti_hbm, zrows_hbm, out_hbm,
                     srcv, dstv, rows, acc, gsem, ssem, *, feat):
    c = lax.axis_index("c")
    s = lax.axis_index("s")
    gwid = c * NS + s

    pltpu.sync_copy(zrows_hbm, acc.at[pl.ds(s * STRIPE, STRIPE)])
    pltpu.sync_copy(srci_hbm.at[gwid], srcv)
    pltpu.sync_copy(dsti_hbm.at[gwid], dstv)
    plsc.subcore_barrier()

    def rowslot(j):
        r = lax.rem(j, RING)
        return rows.at[pl.ds(r * CHUNK, CHUNK)]

    for p in range(LOOK):
        pltpu.async_copy(g_hbm.at[srcv.at[p]], rowslot(p), gsem)

    def body(j, carry):
        q = j + LOOK

        @pl.when(q < NCHUNK)
        def _prefetch():
            @pl.when(j >= RING - LOOK)
            def _():
                pltpu.make_async_copy(rowslot(j), acc.at[dstv.at[j]],
                                      ssem).wait()
            pltpu.async_copy(g_hbm.at[srcv.at[q]], rowslot(q), gsem)

        pltpu.make_async_copy(g_hbm.at[srcv.at[j]], rowslot(j), gsem).wait()
        pltpu.async_copy(rowslot(j), acc.at[dstv.at[j]], ssem, add=True)
        return carry

    lax.fori_loop(0, NCHUNK, body, 0, unroll=False)

    def drain(j, carry):
        pltpu.make_async_copy(rowslot(0), acc.at[dstv.at[0]], ssem).wait()
        return carry

    lax.fori_loop(0, RING, drain, 0, unroll=False)
    plsc.subcore_barrier()
    pltpu.sync_copy(acc.at[pl.ds(s * STRIPE, STRIPE)],
                    out_hbm.at[c].at[pl.ds(s * STRIPE, STRIPE)])


def _make_sc_scatter(feat):
    mesh = plsc.VectorSubcoreMesh(core_axis_name="c", subcore_axis_name="s")
    return functools.partial(
        pl.kernel,
        out_type=jax.ShapeDtypeStruct((NC, N_PAD, feat), jnp.float32),
        mesh=mesh,
        scratch_types=[
            pltpu.VMEM((NCHUNK, CHUNK), jnp.int32),
            pltpu.VMEM((NCHUNK, CHUNK), jnp.int32),
            pltpu.VMEM((RING * CHUNK, feat), jnp.float32),
            pltpu.VMEM_SHARED((N_PAD, feat), jnp.float32),
            pltpu.SemaphoreType.DMA,
            pltpu.SemaphoreType.DMA,
        ],
        compiler_params=_SC_PARAMS,
        name=f"gcn_sc_scatter_{feat}",
    )(functools.partial(_sc_scatter_body, feat=feat))


def _sc_degree_body(ones_hbm, dsti_hbm, zrows_hbm, out_hbm,
                    dstv, rows, acc, sem):
    c = lax.axis_index("c")
    s = lax.axis_index("s")
    gwid = c * NS + s

    pltpu.sync_copy(zrows_hbm, acc.at[pl.ds(s * STRIPE, STRIPE)])
    pltpu.sync_copy(dsti_hbm.at[gwid], dstv)
    pltpu.sync_copy(ones_hbm, rows)
    plsc.subcore_barrier()

    def body(j, carry):
        pltpu.async_copy(rows, acc.at[dstv.at[j]], sem, add=True)
        return carry

    lax.fori_loop(0, NCHUNK, body, 0, unroll=False)

    def drain(j, carry):
        pltpu.make_async_copy(rows, acc.at[dstv.at[0]], sem).wait()
        return carry

    lax.fori_loop(0, NCHUNK, drain, 0, unroll=False)
    plsc.subcore_barrier()
    pltpu.sync_copy(acc.at[pl.ds(s * STRIPE, STRIPE)],
                    out_hbm.at[c].at[pl.ds(s * STRIPE, STRIPE)])


def _make_sc_degree():
    mesh = plsc.VectorSubcoreMesh(core_axis_name="c", subcore_axis_name="s")
    return functools.partial(
        pl.kernel,
        out_type=jax.ShapeDtypeStruct((NC, N_PAD, 16), jnp.float32),
        mesh=mesh,
        scratch_types=[
            pltpu.VMEM((NCHUNK, CHUNK), jnp.int32),
            pltpu.VMEM((CHUNK, 16), jnp.float32),
            pltpu.VMEM_SHARED((N_PAD, 16), jnp.float32),
            pltpu.SemaphoreType.DMA,
        ],
        compiler_params=_SC_PARAMS,
        name="gcn_sc_degree",
    )(_sc_degree_body)


def _tc_matmul_body(x_ref, w_ref, h_ref):
    h_ref[...] = jnp.dot(x_ref[...], w_ref[...],
                         preferred_element_type=jnp.float32)


def _tc_matmul(x, W1):
    return pl.pallas_call(
        _tc_matmul_body,
        out_shape=jax.ShapeDtypeStruct((N, 64), jnp.float32),
    )(x, W1)


def _tc_scale_body(h_ref, deg_ref, g_ref, dinv_ref):
    deg = deg_ref[0, :, 0] + deg_ref[1, :, 0] + 1.0
    dinv = lax.rsqrt(deg)[:, None]
    dinv_ref[...] = dinv
    g_ref[...] = h_ref[...] * dinv[:N]


def _tc_scale(h1, degacc):
    return pl.pallas_call(
        _tc_scale_body,
        out_shape=(jax.ShapeDtypeStruct((N, 64), jnp.float32),
                   jax.ShapeDtypeStruct((N_PAD, 1), jnp.float32)),
    )(h1, degacc)


def _tc_mid_body(acc_ref, g_ref, dinv_ref, b_ref, w_ref, out_ref):
    dinv = dinv_ref[:N]
    z = (acc_ref[0, :N, :] + acc_ref[1, :N, :] + g_ref[...]) * dinv + b_ref[...]
    z = jnp.maximum(z, 0.0)
    out_ref[...] = jnp.dot(z, w_ref[...],
                           preferred_element_type=jnp.float32) * dinv


def _tc_mid(accparts, g_prev, dinv, b, W, h_in, h_out):
    return pl.pallas_call(
        _tc_mid_body,
        out_shape=jax.ShapeDtypeStruct((N, h_out), jnp.float32),
    )(accparts, g_prev, dinv, b.reshape(1, h_in), W)


def _tc_final_body(acc_ref, g_ref, dinv_ref, b3_ref, w4_ref, b4_ref, out_ref):
    dinv = dinv_ref[:N]
    z = (acc_ref[0, :N, :] + acc_ref[1, :N, :] + g_ref[...]) * dinv + b3_ref[...]
    z = jnp.maximum(z, 0.0)
    out_ref[...] = jnp.dot(z, w4_ref[...],
                           preferred_element_type=jnp.float32) + b4_ref[...]


def _tc_final(accparts, g3, dinv, b3, W4, b4):
    return pl.pallas_call(
        _tc_final_body,
        out_shape=jax.ShapeDtypeStruct((N, 2), jnp.float32),
    )(accparts, g3, dinv, b3.reshape(1, 16), W4, b4.reshape(1, 2))


@jax.jit
def kernel(x, edge_index, W1, b1, W2, b2, W3, b3, W4, b4):
    src = edge_index[0].astype(jnp.int32).reshape(NW, NCHUNK, CHUNK)
    dst = edge_index[1].astype(jnp.int32).reshape(NW, NCHUNK, CHUNK)

    ones_rows = jnp.ones((CHUNK, 16), jnp.float32)
    z16 = jnp.zeros((STRIPE, 16), jnp.float32)
    degacc = _make_sc_degree()(ones_rows, dst, z16)

    h1 = _tc_matmul(x, W1)
    g1, dinv = _tc_scale(h1, degacc)
    acc1 = _make_sc_scatter(64)(g1, src, dst, jnp.zeros((STRIPE, 64), jnp.float32))
    g2 = _tc_mid(acc1, g1, dinv, b1, W2, 64, 32)
    acc2 = _make_sc_scatter(32)(g2, src, dst, jnp.zeros((STRIPE, 32), jnp.float32))
    g3 = _tc_mid(acc2, g2, dinv, b2, W3, 32, 16)
    acc3 = _make_sc_scatter(16)(g3, src, dst, jnp.zeros((STRIPE, 16), jnp.float32))
    return _tc_final(acc3, g3, dinv, b3, W4, b4)
